# baseline (speedup 1.0000x reference)
import functools

import jax
import jax.numpy as jnp
from jax import lax
from jax.experimental import pallas as pl
from jax.experimental.pallas import tpu as pltpu
from jax.experimental.pallas import tpu_sc as plsc

N_Z = 16
BATCH = 16384
N_GROUP = 1000000
NC = 2
NS = 16
NW = NC * NS
B_PER_W = BATCH // NW
CH = 16
N_CH = B_PER_W // CH


def _sc_gather_body(idx_hbm, emb1_hbm, emb2_hbm, a_out, b_out,
                    idx_v, sem_a, sem_b):
    wid = lax.axis_index("s") * NC + lax.axis_index("c")
    base = wid * B_PER_W

    pltpu.sync_copy(idx_hbm.at[pl.ds(base, B_PER_W)], idx_v)

    def chunk_body(c, _):
        r0 = base + c * CH
        idx16 = idx_v[pl.ds(c * CH, CH)]
        blk16 = lax.shift_right_logical(idx16, 3)
        sub16 = lax.bitwise_and(idx16, 7)
        for r in range(CH):
            blk = blk16[r]
            sub = sub16[r]
            pltpu.async_copy(
                emb1_hbm.at[blk, sub], a_out.at[r0 + r], sem_a)
            pltpu.async_copy(
                emb2_hbm.at[blk, sub], b_out.at[r0 + r], sem_b)
        return 0

    lax.fori_loop(0, N_CH, chunk_body, 0)

    def drain_body(r, _):
        pltpu.make_async_copy(
            emb1_hbm.at[0, 0], a_out.at[0], sem_a).wait()
        pltpu.make_async_copy(
            emb2_hbm.at[0, 0], b_out.at[0], sem_b).wait()
        return 0

    lax.fori_loop(0, B_PER_W, drain_body, 0)


def _sc_gather(idx, emb1_3, emb2_3):
    mesh = plsc.VectorSubcoreMesh(core_axis_name="c", subcore_axis_name="s")
    k = functools.partial(
        pl.kernel,
        out_type=(
            jax.ShapeDtypeStruct((BATCH, N_Z), jnp.float32),
            jax.ShapeDtypeStruct((BATCH, 1), jnp.float32),
        ),
        mesh=mesh,
        scratch_types=[
            pltpu.VMEM((B_PER_W,), jnp.int32),
            pltpu.SemaphoreType.DMA,
            pltpu.SemaphoreType.DMA,
        ],
    )(_sc_gather_body)
    return k(idx, emb1_3, emb2_3)


def _tc_body(x_ref, a_ref, b_ref, o_ref):
    o_ref[...] = (
        jnp.sum(x_ref[...] * a_ref[...], axis=1, keepdims=True) + b_ref[...]
    )


ROWS_PER_BLK = 2048


def _tc_compute(x, a, b):
    grid = (BATCH // ROWS_PER_BLK,)
    return pl.pallas_call(
        _tc_body,
        grid=grid,
        in_specs=[
            pl.BlockSpec((ROWS_PER_BLK, N_Z), lambda i: (i, 0)),
            pl.BlockSpec((ROWS_PER_BLK, N_Z), lambda i: (i, 0)),
            pl.BlockSpec((ROWS_PER_BLK, 1), lambda i: (i, 0)),
        ],
        out_specs=pl.BlockSpec((ROWS_PER_BLK, 1), lambda i: (i, 0)),
        out_shape=jax.ShapeDtypeStruct((BATCH, 1), jnp.float32),
    )(x, a, b)


@jax.jit
def _rand_effect(x, idx, emb1_3, emb2_3):
    a, b = _sc_gather(idx, emb1_3, emb2_3)
    return _tc_compute(x, a, b)


def kernel(x, idx, emb1, emb2):
    emb1_3 = emb1.reshape(N_GROUP // 8, 8, N_Z)
    emb2_3 = emb2.reshape(N_GROUP // 8, 8, 1)
    return _rand_effect(x, idx.astype(jnp.int32), emb1_3, emb2_3)


# speedup vs baseline: 2.7816x; 2.7816x over previous
import functools

import jax
import jax.numpy as jnp
from jax import lax
from jax.experimental import pallas as pl
from jax.experimental.pallas import tpu as pltpu
from jax.experimental.pallas import tpu_sc as plsc

N_Z = 16
BATCH = 16384
NC = 2
NS = 16
NW = NC * NS
B_PER_W = BATCH // NW
IDX_CHUNK = 128
N_CHUNKS = B_PER_W // IDX_CHUNK
N_TILES = B_PER_W // N_Z


def _sc_body(x_hbm, idx_hbm, emb1_hbm, emb2_hbm, out_hbm,
             idx_v, x_v, a_v, b_v, o_v, sem1, sem2):
    wid = lax.axis_index("s") * NC + lax.axis_index("c")
    base = wid * B_PER_W

    pltpu.sync_copy(idx_hbm.at[pl.ds(base, B_PER_W)], idx_v)

    copies = []
    for g in range(N_CHUNKS):
        sl = pl.ds(g * IDX_CHUNK, IDX_CHUNK)
        copies.append(pltpu.async_copy(
            emb1_hbm.at[idx_v.at[sl]], a_v.at[sl], sem1))
        copies.append(pltpu.async_copy(
            emb2_hbm.at[idx_v.at[sl]], b_v.at[sl], sem2))
    pltpu.sync_copy(x_hbm.at[pl.ds(base, B_PER_W)], x_v)
    for c in copies:
        c.wait()

    cols = [jnp.full((N_Z,), c, jnp.int32) for c in range(N_Z)]

    def tile_body(t, _):
        rows = t * N_Z + lax.iota(jnp.int32, N_Z)
        acc = b_v[pl.ds(t * N_Z, N_Z)]
        for c in range(N_Z):
            xs = plsc.load_gather(x_v, [rows, cols[c]])
            av = plsc.load_gather(a_v, [rows, cols[c]])
            acc = acc + xs * av
        o_v[pl.ds(t * N_Z, N_Z)] = acc
        return 0

    lax.fori_loop(0, N_TILES, tile_body, 0)
    pltpu.sync_copy(o_v, out_hbm.at[pl.ds(base, B_PER_W)])


@jax.jit
def _rand_effect(x, idx, emb1, emb2):
    mesh = plsc.VectorSubcoreMesh(core_axis_name="c", subcore_axis_name="s")
    k = functools.partial(
        pl.kernel,
        out_type=jax.ShapeDtypeStruct((BATCH,), jnp.float32),
        mesh=mesh,
        compiler_params=pltpu.CompilerParams(
            needs_layout_passes=False, use_tc_tiling_on_sc=False),
        scratch_types=[
            pltpu.VMEM((B_PER_W,), jnp.int32),
            pltpu.VMEM((B_PER_W, N_Z), jnp.float32),
            pltpu.VMEM((B_PER_W, N_Z), jnp.float32),
            pltpu.VMEM((B_PER_W,), jnp.float32),
            pltpu.VMEM((B_PER_W,), jnp.float32),
            pltpu.SemaphoreType.DMA,
            pltpu.SemaphoreType.DMA,
        ],
    )(_sc_body)
    return k(x, idx, emb1, emb2)


def kernel(x, idx, emb1, emb2):
    out = _rand_effect(x, idx.astype(jnp.int32), emb1, emb2.reshape(-1))
    return out.reshape(BATCH, 1)
